# Initial kernel scaffold; baseline (speedup 1.0000x reference)
import functools
import jax
import jax.numpy as jnp
from jax import lax
from jax.experimental import pallas as pl
from jax.experimental.pallas import tpu as pltpu
from jax.experimental.pallas import tpu_sc as plsc

N = 50000
D = 128
Q = 32
NQ = 4
NC, NS = 2, 16
NW = NC * NS
E_PAD = 307200
TILE_E = E_PAD // NS
WORK_E = E_PAD // NW
CH = 128
TILE_CH = TILE_E // CH
WORK_CH = WORK_E // CH
N_PAD = 50048
ZROWS = 782
ROWS_PER_TILE = N_PAD // NS
OUT_ROWS = N // NS
DUMMY = N
NB = 1000
GRID = N // NB

_mesh = plsc.VectorSubcoreMesh(
    core_axis_name="c", subcore_axis_name="s", num_cores=NC, num_subcores=NS)


@functools.partial(
    pl.kernel,
    out_type=jax.ShapeDtypeStruct((N, D), jnp.float32),
    mesh=_mesh,
    scratch_types=[
        pltpu.VMEM_SHARED((N_PAD, Q), jnp.float32),
        pltpu.VMEM((2, TILE_E), jnp.int32),
        pltpu.VMEM((2, TILE_CH, CH), jnp.int32),
        pltpu.VMEM((CH,), jnp.int32),
        pltpu.VMEM((CH, Q), jnp.float32),
        pltpu.VMEM((ZROWS, Q), jnp.float32),
        pltpu.SemaphoreType.DMA,
    ],
)
def _msg_pass(m0, m1, srcs, dsts, agg, acc, srcv, dstv, gidx, rows, zbuf, sem):
    c = lax.axis_index("c")
    s = lax.axis_index("s")
    pltpu.sync_copy(srcs.at[0, s], srcv.at[0])
    pltpu.sync_copy(srcs.at[1, s], srcv.at[1])
    pltpu.sync_copy(dsts.at[0, s], dstv.at[0])
    pltpu.sync_copy(dsts.at[1, s], dstv.at[1])

    def _zb(i, carry):
        zbuf[i, pl.ds(0, 16)] = jnp.zeros((16,), jnp.float32)
        zbuf[i, pl.ds(16, 16)] = jnp.zeros((16,), jnp.float32)
        return carry
    lax.fori_loop(0, ZROWS, _zb, 0)

    ms = (m0, m1)
    for p in range(2):
        qq = c * 2 + p
        for k in range(4):
            pltpu.sync_copy(
                zbuf, acc.at[pl.ds(s * ROWS_PER_TILE + k * ZROWS, ZROWS)])
        plsc.subcore_barrier()
        for r in range(2):
            def _chunk(j, carry, r=r, qq=qq):
                base = j * CH
                for g in range(CH // 16):
                    sv = srcv[r, pl.ds(base + g * 16, 16)]
                    gidx[pl.ds(g * 16, 16)] = sv * NQ + qq
                pltpu.async_copy(ms[r].at[gidx], rows, sem).wait()
                pltpu.sync_copy(rows, acc.at[dstv.at[r, j]], add=True)
                return carry
            lax.fori_loop(0, TILE_CH, _chunk, 0)
        plsc.subcore_barrier()
        pltpu.sync_copy(
            acc.at[pl.ds(s * OUT_ROWS, OUT_ROWS)],
            agg.at[pl.ds(s * OUT_ROWS, OUT_ROWS), pl.ds(qq * Q, Q)])
        if p == 0:
            plsc.subcore_barrier()


@functools.partial(
    pl.kernel,
    out_type=jax.ShapeDtypeStruct((NC, N, 8), jnp.float32),
    mesh=_mesh,
    scratch_types=[
        pltpu.VMEM_SHARED((N_PAD, 8), jnp.float32),
        pltpu.VMEM((WORK_CH, CH), jnp.int32),
        pltpu.VMEM((CH, 8), jnp.float32),
        pltpu.SemaphoreType.DMA,
    ],
)
def _attr_agg(attrp, dstp, zeros8, sout, acc, dstv, abuf, sem):
    c = lax.axis_index("c")
    s = lax.axis_index("s")
    w = s * NC + c
    pltpu.sync_copy(dstp.at[w], dstv)
    pltpu.sync_copy(zeros8, acc.at[pl.ds(s * ROWS_PER_TILE, ROWS_PER_TILE)])
    plsc.subcore_barrier()

    def _chunk(j, carry):
        pltpu.sync_copy(attrp.at[pl.ds(w * WORK_E + j * CH, CH)], abuf)
        pltpu.sync_copy(abuf, acc.at[dstv.at[j]], add=True)
        return carry
    lax.fori_loop(0, WORK_CH, _chunk, 0)
    plsc.subcore_barrier()
    pltpu.sync_copy(acc.at[pl.ds(s * OUT_ROWS, OUT_ROWS)],
                    sout.at[c, pl.ds(s * OUT_ROWS, OUT_ROWS)])


def _proj_body(x, win, binp, w0, b0, w1, b1, h, m0, m1):
    hv = jnp.dot(x[...], win[...], preferred_element_type=jnp.float32)
    hv = hv + binp[...]
    h[...] = hv
    m0[...] = jnp.dot(hv, w0[...], preferred_element_type=jnp.float32) + b0[...]
    m1[...] = jnp.dot(hv, w1[...], preferred_element_type=jnp.float32) + b1[...]


def _layer_norm(x, g, b):
    m = jnp.mean(x, axis=-1, keepdims=True)
    d = x - m
    v = jnp.mean(d * d, axis=-1, keepdims=True)
    return g * d * lax.rsqrt(v + 1e-5) + b


def _update_next_body(agg, h, sp, wext, gamma, beta, w0, b0, w1, b1,
                      hout, m0, m1):
    svec = sp[0] + sp[1]
    cterm = jnp.dot(svec, wext[...], preferred_element_type=jnp.float32)
    x = h[...] + jnp.maximum(agg[...] + cterm, 0.0)
    hn = _layer_norm(x, gamma[...], beta[...])
    hout[...] = hn
    m0[...] = jnp.dot(hn, w0[...], preferred_element_type=jnp.float32) + b0[...]
    m1[...] = jnp.dot(hn, w1[...], preferred_element_type=jnp.float32) + b1[...]


def _update_final_body(agg, h, sp, wext, gamma, beta, hout):
    svec = sp[0] + sp[1]
    cterm = jnp.dot(svec, wext[...], preferred_element_type=jnp.float32)
    x = h[...] + jnp.maximum(agg[...] + cterm, 0.0)
    hout[...] = _layer_norm(x, gamma[...], beta[...])


_blk = pl.BlockSpec((NB, D), lambda i: (i, 0))
_wblk = pl.BlockSpec((D, D), lambda i: (0, 0))
_bblk = pl.BlockSpec((1, D), lambda i: (0, 0))
_sblk = pl.BlockSpec((NC, NB, 8), lambda i: (0, i, 0))
_eblk = pl.BlockSpec((8, D), lambda i: (0, 0))
_out = jax.ShapeDtypeStruct((N, D), jnp.float32)

_proj = pl.pallas_call(
    _proj_body,
    grid=(GRID,),
    in_specs=[_blk, _wblk, _bblk, _wblk, _bblk, _wblk, _bblk],
    out_specs=[_blk, _blk, _blk],
    out_shape=[_out, _out, _out],
)

_update_next = pl.pallas_call(
    _update_next_body,
    grid=(GRID,),
    in_specs=[_blk, _blk, _sblk, _eblk, _bblk, _bblk,
              _wblk, _bblk, _wblk, _bblk],
    out_specs=[_blk, _blk, _blk],
    out_shape=[_out, _out, _out],
)

_update_final = pl.pallas_call(
    _update_final_body,
    grid=(GRID,),
    in_specs=[_blk, _blk, _sblk, _eblk, _bblk, _bblk],
    out_specs=_blk,
    out_shape=_out,
)


def kernel(node_feat, edge_index_0, edge_attr_0, edge_index_1, edge_attr_1,
           params):
    f32 = jnp.float32
    x = node_feat.reshape(N, D)
    e = edge_index_0.shape[1]
    pad = E_PAD - e

    def _prep(ei):
        src = jnp.concatenate([ei[0], jnp.zeros((pad,), jnp.int32)])
        dst = jnp.concatenate([ei[1], jnp.full((pad,), DUMMY, jnp.int32)])
        return src.reshape(NS, TILE_E), dst.reshape(NS, TILE_CH, CH)

    s0, d0 = _prep(edge_index_0)
    s1, d1 = _prep(edge_index_1)
    srcs = jnp.stack([s0, s1])
    dsts = jnp.stack([d0, d1])
    dstp = jnp.concatenate(
        [edge_index_0[1], jnp.full((pad,), DUMMY, jnp.int32)]
    ).reshape(NW, WORK_CH, CH)
    attrp = jnp.concatenate([
        jnp.concatenate(
            [edge_attr_0, jnp.ones((e, 1), f32), jnp.zeros((e, 3), f32)], 1),
        jnp.zeros((pad, 8), f32)], axis=0)
    zeros8 = jnp.zeros((ROWS_PER_TILE, 8), f32)

    sp = _attr_agg(attrp, dstp, zeros8)

    p = params
    l0, l1 = p["layers"]
    wext = [jnp.concatenate(
        [l["edge_W"][0], l["edge_b"][0][None, :], jnp.zeros((3, D), f32)], 0)
        for l in (l0, l1)]

    h, m0, m1 = _proj(
        x, p["input_W"], p["input_b"][None], l0["node_W"][0],
        l0["node_b"][0][None], l0["node_W"][1], l0["node_b"][1][None])
    agg = _msg_pass(m0.reshape(N * NQ, Q), m1.reshape(N * NQ, Q), srcs, dsts)
    h, m0, m1 = _update_next(
        agg, h, sp, wext[0], l0["gamma"][None], l0["beta"][None],
        l1["node_W"][0], l1["node_b"][0][None],
        l1["node_W"][1], l1["node_b"][1][None])
    agg = _msg_pass(m0.reshape(N * NQ, Q), m1.reshape(N * NQ, Q), srcs, dsts)
    h = _update_final(
        agg, h, sp, wext[1], l1["gamma"][None], l1["beta"][None])
    return h.reshape(1, N, D)


# speedup vs baseline: 2.1285x; 2.1285x over previous
import functools
import jax
import jax.numpy as jnp
from jax import lax
from jax.experimental import pallas as pl
from jax.experimental.pallas import tpu as pltpu
from jax.experimental.pallas import tpu_sc as plsc

N = 50000
D = 128
Q = 32
NQ = 4
NC, NS = 2, 16
NW = NC * NS
E_PAD = 307200
TILE_E = E_PAD // NS
WORK_E = E_PAD // NW
CH = 128
TILE_CH = TILE_E // CH
WORK_CH = WORK_E // CH
N_PAD = 50048
ZROWS = 391
ROWS_PER_TILE = N_PAD // NS
OUT_ROWS = N // NS
DUMMY = N
NB = 1000
GRID = N // NB

_mesh = plsc.VectorSubcoreMesh(
    core_axis_name="c", subcore_axis_name="s", num_cores=NC, num_subcores=NS)


@functools.partial(
    pl.kernel,
    out_type=jax.ShapeDtypeStruct((NQ, N_PAD, Q), jnp.float32),
    mesh=_mesh,
    scratch_types=[
        pltpu.VMEM_SHARED((N_PAD, Q), jnp.float32),
        pltpu.VMEM((CH,), jnp.int32),
        pltpu.VMEM((CH,), jnp.int32),
        pltpu.VMEM((CH,), jnp.int32),
        pltpu.VMEM((CH, Q), jnp.float32),
        pltpu.VMEM((ZROWS, Q), jnp.float32),
        pltpu.SemaphoreType.DMA,
    ],
    compiler_params=pltpu.CompilerParams(use_tc_tiling_on_sc=False),
)
def _msg_pass(m0, m1, srcs, dsts, agg, acc, srcv, dstv, gidx, rows, zbuf, sem):
    c = lax.axis_index("c")
    s = lax.axis_index("s")

    def _zb(i, carry):
        zbuf[i, pl.ds(0, 16)] = jnp.zeros((16,), jnp.float32)
        zbuf[i, pl.ds(16, 16)] = jnp.zeros((16,), jnp.float32)
        return carry
    lax.fori_loop(0, ZROWS, _zb, 0)

    ms = (m0, m1)
    for p in range(2):
        qq = c * 2 + p
        for k in range(ROWS_PER_TILE // ZROWS):
            pltpu.sync_copy(
                zbuf, acc.at[pl.ds(s * ROWS_PER_TILE + k * ZROWS, ZROWS)])
        plsc.subcore_barrier()
        for r in range(2):
            def _chunk(j, carry, r=r, qq=qq):
                pltpu.sync_copy(srcs.at[r, s, j], srcv)
                pltpu.sync_copy(dsts.at[r, s, j], dstv)
                for g in range(CH // 16):
                    sv = srcv[pl.ds(g * 16, 16)]
                    gidx[pl.ds(g * 16, 16)] = sv * NQ + qq
                pltpu.async_copy(ms[r].at[gidx], rows, sem).wait()
                pltpu.sync_copy(rows, acc.at[dstv], add=True)
                return carry
            lax.fori_loop(0, TILE_CH, _chunk, 0)
        plsc.subcore_barrier()
        pltpu.sync_copy(
            acc.at[pl.ds(s * ROWS_PER_TILE, ROWS_PER_TILE)],
            agg.at[qq, pl.ds(s * ROWS_PER_TILE, ROWS_PER_TILE)])


@functools.partial(
    pl.kernel,
    out_type=jax.ShapeDtypeStruct((NC, N_PAD, 8), jnp.float32),
    mesh=_mesh,
    scratch_types=[
        pltpu.VMEM_SHARED((N_PAD, 8), jnp.float32),
        pltpu.VMEM((WORK_CH, CH), jnp.int32),
        pltpu.VMEM((CH, 8), jnp.float32),
        pltpu.SemaphoreType.DMA,
    ],
    compiler_params=pltpu.CompilerParams(use_tc_tiling_on_sc=False),
)
def _attr_agg(attrp, dstp, zeros8, sout, acc, dstv, abuf, sem):
    c = lax.axis_index("c")
    s = lax.axis_index("s")
    w = s * NC + c
    pltpu.sync_copy(dstp.at[w], dstv)
    pltpu.sync_copy(zeros8, acc.at[pl.ds(s * ROWS_PER_TILE, ROWS_PER_TILE)])
    plsc.subcore_barrier()

    def _chunk(j, carry):
        pltpu.sync_copy(attrp.at[pl.ds(w * WORK_E + j * CH, CH)], abuf)
        pltpu.sync_copy(abuf, acc.at[dstv.at[j]], add=True)
        return carry
    lax.fori_loop(0, WORK_CH, _chunk, 0)
    plsc.subcore_barrier()
    pltpu.sync_copy(acc.at[pl.ds(s * ROWS_PER_TILE, ROWS_PER_TILE)],
                    sout.at[c, pl.ds(s * ROWS_PER_TILE, ROWS_PER_TILE)])


def _proj_body(x, win, binp, w0, b0, w1, b1, h, m0, m1):
    hv = jnp.dot(x[...], win[...], preferred_element_type=jnp.float32)
    hv = hv + binp[...]
    h[...] = hv
    m0[...] = jnp.dot(hv, w0[...], preferred_element_type=jnp.float32) + b0[...]
    m1[...] = jnp.dot(hv, w1[...], preferred_element_type=jnp.float32) + b1[...]


def _layer_norm(x, g, b):
    m = jnp.mean(x, axis=-1, keepdims=True)
    d = x - m
    v = jnp.mean(d * d, axis=-1, keepdims=True)
    return g * d * lax.rsqrt(v + 1e-5) + b


def _update_next_body(agg, h, sp, wext, gamma, beta, w0, b0, w1, b1,
                      hout, m0, m1):
    svec = sp[0] + sp[1]
    cterm = jnp.dot(svec, wext[...], preferred_element_type=jnp.float32)
    aggf = jnp.concatenate([agg[0], agg[1], agg[2], agg[3]], axis=-1)
    x = h[...] + jnp.maximum(aggf + cterm, 0.0)
    hn = _layer_norm(x, gamma[...], beta[...])
    hout[...] = hn
    m0[...] = jnp.dot(hn, w0[...], preferred_element_type=jnp.float32) + b0[...]
    m1[...] = jnp.dot(hn, w1[...], preferred_element_type=jnp.float32) + b1[...]


def _update_final_body(agg, h, sp, wext, gamma, beta, hout):
    svec = sp[0] + sp[1]
    cterm = jnp.dot(svec, wext[...], preferred_element_type=jnp.float32)
    aggf = jnp.concatenate([agg[0], agg[1], agg[2], agg[3]], axis=-1)
    x = h[...] + jnp.maximum(aggf + cterm, 0.0)
    hout[...] = _layer_norm(x, gamma[...], beta[...])


_blk = pl.BlockSpec((NB, D), lambda i: (i, 0))
_wblk = pl.BlockSpec((D, D), lambda i: (0, 0))
_bblk = pl.BlockSpec((1, D), lambda i: (0, 0))
_sblk = pl.BlockSpec((NC, NB, 8), lambda i: (0, i, 0))
_qblk = pl.BlockSpec((NQ, NB, Q), lambda i: (0, i, 0))
_eblk = pl.BlockSpec((8, D), lambda i: (0, 0))
_out = jax.ShapeDtypeStruct((N, D), jnp.float32)

_proj = pl.pallas_call(
    _proj_body,
    grid=(GRID,),
    in_specs=[_blk, _wblk, _bblk, _wblk, _bblk, _wblk, _bblk],
    out_specs=[_blk, _blk, _blk],
    out_shape=[_out, _out, _out],
)

_update_next = pl.pallas_call(
    _update_next_body,
    grid=(GRID,),
    in_specs=[_qblk, _blk, _sblk, _eblk, _bblk, _bblk,
              _wblk, _bblk, _wblk, _bblk],
    out_specs=[_blk, _blk, _blk],
    out_shape=[_out, _out, _out],
)

_update_final = pl.pallas_call(
    _update_final_body,
    grid=(GRID,),
    in_specs=[_qblk, _blk, _sblk, _eblk, _bblk, _bblk],
    out_specs=_blk,
    out_shape=_out,
)


def kernel(node_feat, edge_index_0, edge_attr_0, edge_index_1, edge_attr_1,
           params):
    f32 = jnp.float32
    x = node_feat.reshape(N, D)
    e = edge_index_0.shape[1]
    pad = E_PAD - e

    def _prep(ei):
        src = jnp.concatenate([ei[0], jnp.zeros((pad,), jnp.int32)])
        dst = jnp.concatenate([ei[1], jnp.full((pad,), DUMMY, jnp.int32)])
        return src.reshape(NS, TILE_CH, CH), dst.reshape(NS, TILE_CH, CH)

    s0, d0 = _prep(edge_index_0)
    s1, d1 = _prep(edge_index_1)
    srcs = jnp.stack([s0, s1])
    dsts = jnp.stack([d0, d1])
    dstp = jnp.concatenate(
        [edge_index_0[1], jnp.full((pad,), DUMMY, jnp.int32)]
    ).reshape(NW, WORK_CH, CH)
    attrp = jnp.concatenate([
        jnp.concatenate(
            [edge_attr_0, jnp.ones((e, 1), f32), jnp.zeros((e, 3), f32)], 1),
        jnp.zeros((pad, 8), f32)], axis=0)
    zeros8 = jnp.zeros((ROWS_PER_TILE, 8), f32)

    sp = _attr_agg(attrp, dstp, zeros8)

    p = params
    l0, l1 = p["layers"]
    wext = [jnp.concatenate(
        [l["edge_W"][0], l["edge_b"][0][None, :], jnp.zeros((3, D), f32)], 0)
        for l in (l0, l1)]

    h, m0, m1 = _proj(
        x, p["input_W"], p["input_b"][None], l0["node_W"][0],
        l0["node_b"][0][None], l0["node_W"][1], l0["node_b"][1][None])
    agg = _msg_pass(m0.reshape(N * NQ, Q), m1.reshape(N * NQ, Q), srcs, dsts)
    h, m0, m1 = _update_next(
        agg, h, sp, wext[0], l0["gamma"][None], l0["beta"][None],
        l1["node_W"][0], l1["node_b"][0][None],
        l1["node_W"][1], l1["node_b"][1][None])
    agg = _msg_pass(m0.reshape(N * NQ, Q), m1.reshape(N * NQ, Q), srcs, dsts)
    h = _update_final(
        agg, h, sp, wext[1], l1["gamma"][None], l1["beta"][None])
    return h.reshape(1, N, D)


# speedup vs baseline: 3.0996x; 1.4562x over previous
import functools
import jax
import jax.numpy as jnp
from jax import lax
from jax.experimental import pallas as pl
from jax.experimental.pallas import tpu as pltpu
from jax.experimental.pallas import tpu_sc as plsc

N = 50000
D = 128
Q = 32
NQ = 4
NC, NS = 2, 16
NW = NC * NS
E_PAD = 307200
TILE_E = E_PAD // NS
WORK_E = E_PAD // NW
CH = 128
IB = 10
TILE_CH = TILE_E // CH
WORK_CH = WORK_E // CH
N_PAD = 50048
ZROWS = 391
ROWS_PER_TILE = N_PAD // NS
OUT_ROWS = N // NS
DUMMY = N
NB = 1000
GRID = N // NB

_mesh = plsc.VectorSubcoreMesh(
    core_axis_name="c", subcore_axis_name="s", num_cores=NC, num_subcores=NS)


@functools.partial(
    pl.kernel,
    out_type=jax.ShapeDtypeStruct((NQ, N_PAD, Q), jnp.float32),
    mesh=_mesh,
    scratch_types=[
        pltpu.VMEM_SHARED((N_PAD, Q), jnp.float32),
        pltpu.VMEM((IB, CH), jnp.int32),
        pltpu.VMEM((IB, CH), jnp.int32),
        pltpu.VMEM((2, CH), jnp.int32),
        pltpu.VMEM((2, CH, Q), jnp.float32),
        pltpu.VMEM((ZROWS, Q), jnp.float32),
        pltpu.SemaphoreType.DMA,
    ],
    compiler_params=pltpu.CompilerParams(use_tc_tiling_on_sc=False),
)
def _msg_pass(m0, m1, srcs, dsts, agg, acc, srcblk, dstblk, gidx, rows, zbuf,
              sem):
    c = lax.axis_index("c")
    s = lax.axis_index("s")

    def _zb(i, carry):
        zbuf[i, pl.ds(0, 16)] = jnp.zeros((16,), jnp.float32)
        zbuf[i, pl.ds(16, 16)] = jnp.zeros((16,), jnp.float32)
        return carry
    lax.fori_loop(0, ZROWS, _zb, 0)

    ms = (m0, m1)
    for p in range(2):
        qq = c * 2 + p
        for k in range(ROWS_PER_TILE // ZROWS):
            pltpu.sync_copy(
                zbuf, acc.at[pl.ds(s * ROWS_PER_TILE + k * ZROWS, ZROWS)])
        plsc.subcore_barrier()
        for r in range(2):
            def _blk(ib, carry, r=r, qq=qq):
                pltpu.sync_copy(srcs.at[r, s, pl.ds(ib * IB, IB)], srcblk)
                pltpu.sync_copy(dsts.at[r, s, pl.ds(ib * IB, IB)], dstblk)
                cps = []
                for b in range(IB):
                    for g in range(CH // 16):
                        sv = srcblk[b, pl.ds(g * 16, 16)]
                        gidx[b & 1, pl.ds(g * 16, 16)] = sv * NQ + qq
                    cps.append(pltpu.async_copy(
                        ms[r].at[gidx.at[b & 1]], rows.at[b & 1], sem))
                    if b > 0:
                        cps[b - 1].wait()
                        pltpu.sync_copy(rows.at[(b - 1) & 1],
                                        acc.at[dstblk.at[b - 1]], add=True)
                cps[IB - 1].wait()
                pltpu.sync_copy(rows.at[(IB - 1) & 1],
                                acc.at[dstblk.at[IB - 1]], add=True)
                return carry
            lax.fori_loop(0, TILE_CH // IB, _blk, 0)
        plsc.subcore_barrier()
        pltpu.sync_copy(
            acc.at[pl.ds(s * ROWS_PER_TILE, ROWS_PER_TILE)],
            agg.at[qq, pl.ds(s * ROWS_PER_TILE, ROWS_PER_TILE)])


@functools.partial(
    pl.kernel,
    out_type=jax.ShapeDtypeStruct((NC, N_PAD, 8), jnp.float32),
    mesh=_mesh,
    scratch_types=[
        pltpu.VMEM_SHARED((N_PAD, 8), jnp.float32),
        pltpu.VMEM((WORK_CH, CH), jnp.int32),
        pltpu.VMEM((CH, 8), jnp.float32),
        pltpu.SemaphoreType.DMA,
    ],
    compiler_params=pltpu.CompilerParams(use_tc_tiling_on_sc=False),
)
def _attr_agg(attrp, dstp, zeros8, sout, acc, dstv, abuf, sem):
    c = lax.axis_index("c")
    s = lax.axis_index("s")
    w = s * NC + c
    pltpu.sync_copy(dstp.at[w], dstv)
    pltpu.sync_copy(zeros8, acc.at[pl.ds(s * ROWS_PER_TILE, ROWS_PER_TILE)])
    plsc.subcore_barrier()

    def _chunk(j, carry):
        pltpu.sync_copy(attrp.at[pl.ds(w * WORK_E + j * CH, CH)], abuf)
        pltpu.sync_copy(abuf, acc.at[dstv.at[j]], add=True)
        return carry
    lax.fori_loop(0, WORK_CH, _chunk, 0)
    plsc.subcore_barrier()
    pltpu.sync_copy(acc.at[pl.ds(s * ROWS_PER_TILE, ROWS_PER_TILE)],
                    sout.at[c, pl.ds(s * ROWS_PER_TILE, ROWS_PER_TILE)])


def _proj_body(x, win, binp, w0, b0, w1, b1, h, m0, m1):
    hv = jnp.dot(x[...], win[...], preferred_element_type=jnp.float32)
    hv = hv + binp[...]
    h[...] = hv
    m0[...] = jnp.dot(hv, w0[...], preferred_element_type=jnp.float32) + b0[...]
    m1[...] = jnp.dot(hv, w1[...], preferred_element_type=jnp.float32) + b1[...]


def _layer_norm(x, g, b):
    m = jnp.mean(x, axis=-1, keepdims=True)
    d = x - m
    v = jnp.mean(d * d, axis=-1, keepdims=True)
    return g * d * lax.rsqrt(v + 1e-5) + b


def _update_next_body(agg, h, sp, wext, gamma, beta, w0, b0, w1, b1,
                      hout, m0, m1):
    svec = sp[0] + sp[1]
    cterm = jnp.dot(svec, wext[...], preferred_element_type=jnp.float32)
    aggf = jnp.concatenate([agg[0], agg[1], agg[2], agg[3]], axis=-1)
    x = h[...] + jnp.maximum(aggf + cterm, 0.0)
    hn = _layer_norm(x, gamma[...], beta[...])
    hout[...] = hn
    m0[...] = jnp.dot(hn, w0[...], preferred_element_type=jnp.float32) + b0[...]
    m1[...] = jnp.dot(hn, w1[...], preferred_element_type=jnp.float32) + b1[...]


def _update_final_body(agg, h, sp, wext, gamma, beta, hout):
    svec = sp[0] + sp[1]
    cterm = jnp.dot(svec, wext[...], preferred_element_type=jnp.float32)
    aggf = jnp.concatenate([agg[0], agg[1], agg[2], agg[3]], axis=-1)
    x = h[...] + jnp.maximum(aggf + cterm, 0.0)
    hout[...] = _layer_norm(x, gamma[...], beta[...])


_blk = pl.BlockSpec((NB, D), lambda i: (i, 0))
_wblk = pl.BlockSpec((D, D), lambda i: (0, 0))
_bblk = pl.BlockSpec((1, D), lambda i: (0, 0))
_sblk = pl.BlockSpec((NC, NB, 8), lambda i: (0, i, 0))
_qblk = pl.BlockSpec((NQ, NB, Q), lambda i: (0, i, 0))
_eblk = pl.BlockSpec((8, D), lambda i: (0, 0))
_out = jax.ShapeDtypeStruct((N, D), jnp.float32)

_proj = pl.pallas_call(
    _proj_body,
    grid=(GRID,),
    in_specs=[_blk, _wblk, _bblk, _wblk, _bblk, _wblk, _bblk],
    out_specs=[_blk, _blk, _blk],
    out_shape=[_out, _out, _out],
)

_update_next = pl.pallas_call(
    _update_next_body,
    grid=(GRID,),
    in_specs=[_qblk, _blk, _sblk, _eblk, _bblk, _bblk,
              _wblk, _bblk, _wblk, _bblk],
    out_specs=[_blk, _blk, _blk],
    out_shape=[_out, _out, _out],
)

_update_final = pl.pallas_call(
    _update_final_body,
    grid=(GRID,),
    in_specs=[_qblk, _blk, _sblk, _eblk, _bblk, _bblk],
    out_specs=_blk,
    out_shape=_out,
)


def kernel(node_feat, edge_index_0, edge_attr_0, edge_index_1, edge_attr_1,
           params):
    f32 = jnp.float32
    x = node_feat.reshape(N, D)
    e = edge_index_0.shape[1]
    pad = E_PAD - e

    def _prep(ei):
        src = jnp.concatenate([ei[0], jnp.zeros((pad,), jnp.int32)])
        dst = jnp.concatenate([ei[1], jnp.full((pad,), DUMMY, jnp.int32)])
        return src.reshape(NS, TILE_CH, CH), dst.reshape(NS, TILE_CH, CH)

    s0, d0 = _prep(edge_index_0)
    s1, d1 = _prep(edge_index_1)
    srcs = jnp.stack([s0, s1])
    dsts = jnp.stack([d0, d1])
    dstp = jnp.concatenate(
        [edge_index_0[1], jnp.full((pad,), DUMMY, jnp.int32)]
    ).reshape(NW, WORK_CH, CH)
    attrp = jnp.concatenate([
        jnp.concatenate(
            [edge_attr_0, jnp.ones((e, 1), f32), jnp.zeros((e, 3), f32)], 1),
        jnp.zeros((pad, 8), f32)], axis=0)
    zeros8 = jnp.zeros((ROWS_PER_TILE, 8), f32)

    sp = _attr_agg(attrp, dstp, zeros8)

    p = params
    l0, l1 = p["layers"]
    wext = [jnp.concatenate(
        [l["edge_W"][0], l["edge_b"][0][None, :], jnp.zeros((3, D), f32)], 0)
        for l in (l0, l1)]

    h, m0, m1 = _proj(
        x, p["input_W"], p["input_b"][None], l0["node_W"][0],
        l0["node_b"][0][None], l0["node_W"][1], l0["node_b"][1][None])
    agg = _msg_pass(m0.reshape(N * NQ, Q), m1.reshape(N * NQ, Q), srcs, dsts)
    h, m0, m1 = _update_next(
        agg, h, sp, wext[0], l0["gamma"][None], l0["beta"][None],
        l1["node_W"][0], l1["node_b"][0][None],
        l1["node_W"][1], l1["node_b"][1][None])
    agg = _msg_pass(m0.reshape(N * NQ, Q), m1.reshape(N * NQ, Q), srcs, dsts)
    h = _update_final(
        agg, h, sp, wext[1], l1["gamma"][None], l1["beta"][None])
    return h.reshape(1, N, D)


# speedup vs baseline: 3.2279x; 1.0414x over previous
import functools
import jax
import jax.numpy as jnp
from jax import lax
from jax.experimental import pallas as pl
from jax.experimental.pallas import tpu as pltpu
from jax.experimental.pallas import tpu_sc as plsc

N = 50000
D = 128
Q = 32
NQ = 4
NC, NS = 2, 16
NW = NC * NS
E_PAD = 307200
TILE_E = E_PAD // NS
WORK_E = E_PAD // NW
CH = 128
IB = 10
TILE_CH = TILE_E // CH
WORK_CH = WORK_E // CH
N_PAD = 50048
ZROWS = 136
ROWS_PER_TILE = N_PAD // NS
OUT_ROWS = N // NS
DUMMY = N
NB = 1000
GRID = N // NB

_mesh = plsc.VectorSubcoreMesh(
    core_axis_name="c", subcore_axis_name="s", num_cores=NC, num_subcores=NS)


@functools.partial(
    pl.kernel,
    out_type=jax.ShapeDtypeStruct((NQ, N_PAD, Q), jnp.float32),
    mesh=_mesh,
    scratch_types=[
        pltpu.VMEM_SHARED((N_PAD, Q), jnp.float32),
        pltpu.VMEM((IB, CH), jnp.int32),
        pltpu.VMEM((IB, CH), jnp.int32),
        pltpu.VMEM((4, CH), jnp.int32),
        pltpu.VMEM((4, CH, Q), jnp.float32),
        pltpu.VMEM((ZROWS, Q), jnp.float32),
        pltpu.SemaphoreType.DMA,
        pltpu.SemaphoreType.DMA,
    ],
    compiler_params=pltpu.CompilerParams(use_tc_tiling_on_sc=False),
)
def _msg_pass(m0, m1, srcs, dsts, agg, acc, srcblk, dstblk, gidx, rows, zbuf,
              sem, sem2):
    c = lax.axis_index("c")
    s = lax.axis_index("s")

    def _zb(i, carry):
        zbuf[i, pl.ds(0, 16)] = jnp.zeros((16,), jnp.float32)
        zbuf[i, pl.ds(16, 16)] = jnp.zeros((16,), jnp.float32)
        return carry
    lax.fori_loop(0, ZROWS, _zb, 0)

    ms = (m0, m1)
    for p in range(2):
        qq = c * 2 + p
        for k in range(ROWS_PER_TILE // ZROWS):
            pltpu.sync_copy(
                zbuf, acc.at[pl.ds(s * ROWS_PER_TILE + k * ZROWS, ZROWS)])
        plsc.subcore_barrier()
        for r in range(2):
            def _blk(ib, carry, r=r, qq=qq):
                pltpu.sync_copy(srcs.at[r, s, pl.ds(ib * IB, IB)], srcblk)
                pltpu.sync_copy(dsts.at[r, s, pl.ds(ib * IB, IB)], dstblk)
                cps = [None] * IB
                scs = [None] * IB
                for b in range(IB):
                    if b >= 4:
                        scs[b - 4].wait()
                    for g in range(CH // 16):
                        sv = srcblk[b, pl.ds(g * 16, 16)]
                        gidx[b % 4, pl.ds(g * 16, 16)] = sv * NQ + qq
                    cps[b] = pltpu.async_copy(
                        ms[r].at[gidx.at[b % 4]], rows.at[b % 4], sem)
                    if b >= 2:
                        cps[b - 2].wait()
                        scs[b - 2] = pltpu.async_copy(
                            rows.at[(b - 2) % 4],
                            acc.at[dstblk.at[b - 2]], sem2, add=True)
                for b in (IB - 2, IB - 1):
                    cps[b].wait()
                    scs[b] = pltpu.async_copy(
                        rows.at[b % 4], acc.at[dstblk.at[b]], sem2, add=True)
                for b in range(IB - 4, IB):
                    scs[b].wait()
                return carry
            lax.fori_loop(0, TILE_CH // IB, _blk, 0)
        plsc.subcore_barrier()
        pltpu.sync_copy(
            acc.at[pl.ds(s * ROWS_PER_TILE, ROWS_PER_TILE)],
            agg.at[qq, pl.ds(s * ROWS_PER_TILE, ROWS_PER_TILE)])


@functools.partial(
    pl.kernel,
    out_type=jax.ShapeDtypeStruct((NC, N_PAD, 8), jnp.float32),
    mesh=_mesh,
    scratch_types=[
        pltpu.VMEM_SHARED((N_PAD, 8), jnp.float32),
        pltpu.VMEM((WORK_CH, CH), jnp.int32),
        pltpu.VMEM((CH, 8), jnp.float32),
        pltpu.SemaphoreType.DMA,
    ],
    compiler_params=pltpu.CompilerParams(use_tc_tiling_on_sc=False),
)
def _attr_agg(attrp, dstp, zeros8, sout, acc, dstv, abuf, sem):
    c = lax.axis_index("c")
    s = lax.axis_index("s")
    w = s * NC + c
    pltpu.sync_copy(dstp.at[w], dstv)
    pltpu.sync_copy(zeros8, acc.at[pl.ds(s * ROWS_PER_TILE, ROWS_PER_TILE)])
    plsc.subcore_barrier()

    def _chunk(j, carry):
        pltpu.sync_copy(attrp.at[pl.ds(w * WORK_E + j * CH, CH)], abuf)
        pltpu.sync_copy(abuf, acc.at[dstv.at[j]], add=True)
        return carry
    lax.fori_loop(0, WORK_CH, _chunk, 0)
    plsc.subcore_barrier()
    pltpu.sync_copy(acc.at[pl.ds(s * ROWS_PER_TILE, ROWS_PER_TILE)],
                    sout.at[c, pl.ds(s * ROWS_PER_TILE, ROWS_PER_TILE)])


def _proj_body(x, win, binp, w0, b0, w1, b1, h, m0, m1):
    hv = jnp.dot(x[...], win[...], preferred_element_type=jnp.float32)
    hv = hv + binp[...]
    h[...] = hv
    m0[...] = jnp.dot(hv, w0[...], preferred_element_type=jnp.float32) + b0[...]
    m1[...] = jnp.dot(hv, w1[...], preferred_element_type=jnp.float32) + b1[...]


def _layer_norm(x, g, b):
    m = jnp.mean(x, axis=-1, keepdims=True)
    d = x - m
    v = jnp.mean(d * d, axis=-1, keepdims=True)
    return g * d * lax.rsqrt(v + 1e-5) + b


def _update_next_body(agg, h, sp, wext, gamma, beta, w0, b0, w1, b1,
                      hout, m0, m1):
    svec = sp[0] + sp[1]
    cterm = jnp.dot(svec, wext[...], preferred_element_type=jnp.float32)
    aggf = jnp.concatenate([agg[0], agg[1], agg[2], agg[3]], axis=-1)
    x = h[...] + jnp.maximum(aggf + cterm, 0.0)
    hn = _layer_norm(x, gamma[...], beta[...])
    hout[...] = hn
    m0[...] = jnp.dot(hn, w0[...], preferred_element_type=jnp.float32) + b0[...]
    m1[...] = jnp.dot(hn, w1[...], preferred_element_type=jnp.float32) + b1[...]


def _update_final_body(agg, h, sp, wext, gamma, beta, hout):
    svec = sp[0] + sp[1]
    cterm = jnp.dot(svec, wext[...], preferred_element_type=jnp.float32)
    aggf = jnp.concatenate([agg[0], agg[1], agg[2], agg[3]], axis=-1)
    x = h[...] + jnp.maximum(aggf + cterm, 0.0)
    hout[...] = _layer_norm(x, gamma[...], beta[...])


_blk = pl.BlockSpec((NB, D), lambda i: (i, 0))
_wblk = pl.BlockSpec((D, D), lambda i: (0, 0))
_bblk = pl.BlockSpec((1, D), lambda i: (0, 0))
_sblk = pl.BlockSpec((NC, NB, 8), lambda i: (0, i, 0))
_qblk = pl.BlockSpec((NQ, NB, Q), lambda i: (0, i, 0))
_eblk = pl.BlockSpec((8, D), lambda i: (0, 0))
_out = jax.ShapeDtypeStruct((N, D), jnp.float32)

_proj = pl.pallas_call(
    _proj_body,
    grid=(GRID,),
    in_specs=[_blk, _wblk, _bblk, _wblk, _bblk, _wblk, _bblk],
    out_specs=[_blk, _blk, _blk],
    out_shape=[_out, _out, _out],
)

_update_next = pl.pallas_call(
    _update_next_body,
    grid=(GRID,),
    in_specs=[_qblk, _blk, _sblk, _eblk, _bblk, _bblk,
              _wblk, _bblk, _wblk, _bblk],
    out_specs=[_blk, _blk, _blk],
    out_shape=[_out, _out, _out],
)

_update_final = pl.pallas_call(
    _update_final_body,
    grid=(GRID,),
    in_specs=[_qblk, _blk, _sblk, _eblk, _bblk, _bblk],
    out_specs=_blk,
    out_shape=_out,
)


def kernel(node_feat, edge_index_0, edge_attr_0, edge_index_1, edge_attr_1,
           params):
    f32 = jnp.float32
    x = node_feat.reshape(N, D)
    e = edge_index_0.shape[1]
    pad = E_PAD - e

    def _prep(ei):
        src = jnp.concatenate([ei[0], jnp.zeros((pad,), jnp.int32)])
        dst = jnp.concatenate([ei[1], jnp.full((pad,), DUMMY, jnp.int32)])
        return src.reshape(NS, TILE_CH, CH), dst.reshape(NS, TILE_CH, CH)

    s0, d0 = _prep(edge_index_0)
    s1, d1 = _prep(edge_index_1)
    srcs = jnp.stack([s0, s1])
    dsts = jnp.stack([d0, d1])
    dstp = jnp.concatenate(
        [edge_index_0[1], jnp.full((pad,), DUMMY, jnp.int32)]
    ).reshape(NW, WORK_CH, CH)
    attrp = jnp.concatenate([
        jnp.concatenate(
            [edge_attr_0, jnp.ones((e, 1), f32), jnp.zeros((e, 3), f32)], 1),
        jnp.zeros((pad, 8), f32)], axis=0)
    zeros8 = jnp.zeros((ROWS_PER_TILE, 8), f32)

    sp = _attr_agg(attrp, dstp, zeros8)

    p = params
    l0, l1 = p["layers"]
    wext = [jnp.concatenate(
        [l["edge_W"][0], l["edge_b"][0][None, :], jnp.zeros((3, D), f32)], 0)
        for l in (l0, l1)]

    h, m0, m1 = _proj(
        x, p["input_W"], p["input_b"][None], l0["node_W"][0],
        l0["node_b"][0][None], l0["node_W"][1], l0["node_b"][1][None])
    agg = _msg_pass(m0.reshape(N * NQ, Q), m1.reshape(N * NQ, Q), srcs, dsts)
    h, m0, m1 = _update_next(
        agg, h, sp, wext[0], l0["gamma"][None], l0["beta"][None],
        l1["node_W"][0], l1["node_b"][0][None],
        l1["node_W"][1], l1["node_b"][1][None])
    agg = _msg_pass(m0.reshape(N * NQ, Q), m1.reshape(N * NQ, Q), srcs, dsts)
    h = _update_final(
        agg, h, sp, wext[1], l1["gamma"][None], l1["beta"][None])
    return h.reshape(1, N, D)


# speedup vs baseline: 3.8487x; 1.1923x over previous
import functools
import jax
import jax.numpy as jnp
from jax import lax
from jax.experimental import pallas as pl
from jax.experimental.pallas import tpu as pltpu
from jax.experimental.pallas import tpu_sc as plsc

N = 50000
D = 128
NC, NS = 2, 16
NW = NC * NS
CH = 128
STRIPE = 2048
NSTRIPE = 25
N_OUT = NSTRIPE * STRIPE
E_SRT = 300032
DUMMY = STRIPE
TROWS = STRIPE // NS

E_PAD = 307200
WORK_E = E_PAD // NW
WORK_CH = WORK_E // CH
N_PAD = 50048
ROWS_PER_TILE = N_PAD // NS

NB = 1000
GRID = N // NB

_mesh = plsc.VectorSubcoreMesh(
    core_axis_name="c", subcore_axis_name="s", num_cores=NC, num_subcores=NS)


@functools.partial(
    pl.kernel,
    out_type=jax.ShapeDtypeStruct((N_OUT, D), jnp.float32),
    mesh=_mesh,
    scratch_types=[
        pltpu.VMEM_SHARED((STRIPE + 8, D), jnp.float32),
        pltpu.VMEM((2, CH), jnp.int32),
        pltpu.VMEM((2, CH), jnp.int32),
        pltpu.VMEM((CH,), jnp.int32),
        pltpu.VMEM((2, CH, D), jnp.float32),
        pltpu.VMEM((TROWS, D), jnp.float32),
        pltpu.VMEM((2, 48), jnp.int32),
        pltpu.SemaphoreType.DMA,
    ],
    compiler_params=pltpu.CompilerParams(use_tc_tiling_on_sc=False),
)
def _msg_pass(m0, m1, ssrc0, sdst0, ssrc1, sdst1, rp, agg,
              acc, srcc, dstc, dloc, rows, zbuf, rpv, sem):
    c = lax.axis_index("c")
    s = lax.axis_index("s")
    pltpu.sync_copy(rp, rpv)
    ms = (m0, m1)
    srcs = (ssrc0, ssrc1)
    dsts = (sdst0, sdst1)

    def _zb(i, carry):
        for g in range(D // 16):
            zbuf[i, pl.ds(g * 16, 16)] = jnp.zeros((16,), jnp.float32)
        return carry
    lax.fori_loop(0, TROWS, _zb, 0)

    def _stripe(sid):
        lo = sid * STRIPE
        pltpu.sync_copy(zbuf, acc.at[pl.ds(s * TROWS, TROWS)])

        @pl.when(s == 0)
        def _():
            pltpu.sync_copy(zbuf.at[pl.ds(0, 8)], acc.at[pl.ds(STRIPE, 8)])
        plsc.subcore_barrier()

        for r in range(2):
            rpvec = rpv[r, pl.ds(sid, 16)]
            p0 = rpvec[0]
            p1 = rpvec[1]
            a0 = (p0 // CH) * CH
            nch = (p1 - a0 + CH - 1) // CH
            nmy = jnp.maximum(nch - s + NS - 1, 0) // NS

            def _fire(t, b, r=r, a0=a0):
                off = a0 + (s + t * NS) * CH
                pltpu.sync_copy(srcs[r].at[pl.ds(off, CH)], srcc.at[b])
                pltpu.sync_copy(dsts[r].at[pl.ds(off, CH)], dstc.at[b])
                return pltpu.async_copy(ms[r].at[srcc.at[b]], rows.at[b], sem)

            @pl.when(nmy > 0)
            def _():
                _fire(0, 0)

            def _ch(t, carry, r=r, nmy=nmy, lo=lo):
                b = lax.rem(t, 2)

                @pl.when(t + 1 < nmy)
                def _():
                    _fire(t + 1, 1 - b)
                pltpu.make_async_copy(ms[r].at[srcc.at[b]], rows.at[b],
                                      sem).wait()
                for g in range(CH // 16):
                    u = dstc[b, pl.ds(g * 16, 16)] - lo
                    bad = (u < 0) | (u >= STRIPE)
                    dloc[pl.ds(g * 16, 16)] = jnp.where(bad, DUMMY, u)
                pltpu.sync_copy(rows.at[b], acc.at[dloc], add=True)
                return carry
            lax.fori_loop(0, nmy, _ch, 0)

        plsc.subcore_barrier()
        pltpu.sync_copy(acc.at[pl.ds(s * TROWS, TROWS)],
                        agg.at[pl.ds(lo + s * TROWS, TROWS)])

    for k in range(NSTRIPE // NC):
        _stripe(NC * k + c)

    @pl.when(NC * (NSTRIPE // NC) + c < NSTRIPE)
    def _():
        _stripe(NC * (NSTRIPE // NC) + c)


@functools.partial(
    pl.kernel,
    out_type=jax.ShapeDtypeStruct((NC, N_PAD, 8), jnp.float32),
    mesh=_mesh,
    scratch_types=[
        pltpu.VMEM_SHARED((N_PAD, 8), jnp.float32),
        pltpu.VMEM((WORK_CH, CH), jnp.int32),
        pltpu.VMEM((CH, 8), jnp.float32),
        pltpu.SemaphoreType.DMA,
    ],
    compiler_params=pltpu.CompilerParams(use_tc_tiling_on_sc=False),
)
def _attr_agg(attrp, dstp, zeros8, sout, acc, dstv, abuf, sem):
    c = lax.axis_index("c")
    s = lax.axis_index("s")
    w = s * NC + c
    pltpu.sync_copy(dstp.at[w], dstv)
    pltpu.sync_copy(zeros8, acc.at[pl.ds(s * ROWS_PER_TILE, ROWS_PER_TILE)])
    plsc.subcore_barrier()

    def _chunk(j, carry):
        pltpu.sync_copy(attrp.at[pl.ds(w * WORK_E + j * CH, CH)], abuf)
        pltpu.sync_copy(abuf, acc.at[dstv.at[j]], add=True)
        return carry
    lax.fori_loop(0, WORK_CH, _chunk, 0)
    plsc.subcore_barrier()
    pltpu.sync_copy(acc.at[pl.ds(s * ROWS_PER_TILE, ROWS_PER_TILE)],
                    sout.at[c, pl.ds(s * ROWS_PER_TILE, ROWS_PER_TILE)])


def _proj_body(x, win, binp, w0, b0, w1, b1, h, m0, m1):
    hv = jnp.dot(x[...], win[...], preferred_element_type=jnp.float32)
    hv = hv + binp[...]
    h[...] = hv
    m0[...] = jnp.dot(hv, w0[...], preferred_element_type=jnp.float32) + b0[...]
    m1[...] = jnp.dot(hv, w1[...], preferred_element_type=jnp.float32) + b1[...]


def _layer_norm(x, g, b):
    m = jnp.mean(x, axis=-1, keepdims=True)
    d = x - m
    v = jnp.mean(d * d, axis=-1, keepdims=True)
    return g * d * lax.rsqrt(v + 1e-5) + b


def _update_next_body(agg, h, sp, wext, gamma, beta, w0, b0, w1, b1,
                      hout, m0, m1):
    svec = sp[0] + sp[1]
    cterm = jnp.dot(svec, wext[...], preferred_element_type=jnp.float32)
    x = h[...] + jnp.maximum(agg[...] + cterm, 0.0)
    hn = _layer_norm(x, gamma[...], beta[...])
    hout[...] = hn
    m0[...] = jnp.dot(hn, w0[...], preferred_element_type=jnp.float32) + b0[...]
    m1[...] = jnp.dot(hn, w1[...], preferred_element_type=jnp.float32) + b1[...]


def _update_final_body(agg, h, sp, wext, gamma, beta, hout):
    svec = sp[0] + sp[1]
    cterm = jnp.dot(svec, wext[...], preferred_element_type=jnp.float32)
    x = h[...] + jnp.maximum(agg[...] + cterm, 0.0)
    hout[...] = _layer_norm(x, gamma[...], beta[...])


_blk = pl.BlockSpec((NB, D), lambda i: (i, 0))
_wblk = pl.BlockSpec((D, D), lambda i: (0, 0))
_bblk = pl.BlockSpec((1, D), lambda i: (0, 0))
_sblk = pl.BlockSpec((NC, NB, 8), lambda i: (0, i, 0))
_eblk = pl.BlockSpec((8, D), lambda i: (0, 0))
_out = jax.ShapeDtypeStruct((N, D), jnp.float32)

_proj = pl.pallas_call(
    _proj_body,
    grid=(GRID,),
    in_specs=[_blk, _wblk, _bblk, _wblk, _bblk, _wblk, _bblk],
    out_specs=[_blk, _blk, _blk],
    out_shape=[_out, _out, _out],
)

_update_next = pl.pallas_call(
    _update_next_body,
    grid=(GRID,),
    in_specs=[_blk, _blk, _sblk, _eblk, _bblk, _bblk,
              _wblk, _bblk, _wblk, _bblk],
    out_specs=[_blk, _blk, _blk],
    out_shape=[_out, _out, _out],
)

_update_final = pl.pallas_call(
    _update_final_body,
    grid=(GRID,),
    in_specs=[_blk, _blk, _sblk, _eblk, _bblk, _bblk],
    out_specs=_blk,
    out_shape=_out,
)


def kernel(node_feat, edge_index_0, edge_attr_0, edge_index_1, edge_attr_1,
           params):
    f32 = jnp.float32
    x = node_feat.reshape(N, D)
    e = edge_index_0.shape[1]

    bounds = jnp.arange(NSTRIPE + 1, dtype=jnp.int32) * STRIPE

    def _sort_rel(ei):
        sdst, ssrc = lax.sort([ei[1], ei[0]], num_keys=1)
        rp = jnp.searchsorted(sdst, bounds).astype(jnp.int32)
        spad = E_SRT - e
        ssrc = jnp.concatenate([ssrc, jnp.zeros((spad,), jnp.int32)])
        sdst = jnp.concatenate(
            [sdst, jnp.full((spad,), jnp.int32(2 ** 24), jnp.int32)])
        rp = jnp.concatenate(
            [rp, jnp.zeros((48 - NSTRIPE - 1,), jnp.int32)])
        return ssrc, sdst, rp

    ssrc0, sdst0, rp0 = _sort_rel(edge_index_0)
    ssrc1, sdst1, rp1 = _sort_rel(edge_index_1)
    rp = jnp.stack([rp0, rp1])

    pad = E_PAD - e
    dstp = jnp.concatenate(
        [edge_index_0[1], jnp.full((pad,), N, jnp.int32)]
    ).reshape(NW, WORK_CH, CH)
    attrp = jnp.concatenate([
        jnp.concatenate(
            [edge_attr_0, jnp.ones((e, 1), f32), jnp.zeros((e, 3), f32)], 1),
        jnp.zeros((pad, 8), f32)], axis=0)
    zeros8 = jnp.zeros((ROWS_PER_TILE, 8), f32)

    sp = _attr_agg(attrp, dstp, zeros8)

    p = params
    l0, l1 = p["layers"]
    wext = [jnp.concatenate(
        [l["edge_W"][0], l["edge_b"][0][None, :], jnp.zeros((3, D), f32)], 0)
        for l in (l0, l1)]

    h, m0, m1 = _proj(
        x, p["input_W"], p["input_b"][None], l0["node_W"][0],
        l0["node_b"][0][None], l0["node_W"][1], l0["node_b"][1][None])
    agg = _msg_pass(m0, m1, ssrc0, sdst0, ssrc1, sdst1, rp)
    h, m0, m1 = _update_next(
        agg, h, sp, wext[0], l0["gamma"][None], l0["beta"][None],
        l1["node_W"][0], l1["node_b"][0][None],
        l1["node_W"][1], l1["node_b"][1][None])
    agg = _msg_pass(m0, m1, ssrc0, sdst0, ssrc1, sdst1, rp)
    h = _update_final(
        agg, h, sp, wext[1], l1["gamma"][None], l1["beta"][None])
    return h.reshape(1, N, D)


# speedup vs baseline: 4.1827x; 1.0868x over previous
import functools
import jax
import jax.numpy as jnp
from jax import lax
from jax.experimental import pallas as pl
from jax.experimental.pallas import tpu as pltpu
from jax.experimental.pallas import tpu_sc as plsc

N = 50000
D = 128
NC, NS = 2, 16
NW = NC * NS
CH = 128
STRIPE = 2048
NSTRIPE = 25
N_OUT = NSTRIPE * STRIPE
E_SRT = 300032
DUMMY = STRIPE
TROWS = STRIPE // NS

E_PAD = 307200
WORK_E = E_PAD // NW
WORK_CH = WORK_E // CH
N_PAD = 50048
ROWS_PER_TILE = N_PAD // NS

NB = 1000
GRID = N // NB

_mesh = plsc.VectorSubcoreMesh(
    core_axis_name="c", subcore_axis_name="s", num_cores=NC, num_subcores=NS)


@functools.partial(
    pl.kernel,
    out_type=jax.ShapeDtypeStruct((N_OUT, D), jnp.float32),
    mesh=_mesh,
    scratch_types=[
        pltpu.VMEM_SHARED((STRIPE + 8, D), jnp.float32),
        pltpu.VMEM((2, CH), jnp.int32),
        pltpu.VMEM((2, CH), jnp.int32),
        pltpu.VMEM((CH,), jnp.int32),
        pltpu.VMEM((2, CH, D), jnp.float32),
        pltpu.VMEM((TROWS, D), jnp.float32),
        pltpu.VMEM((2, 48), jnp.int32),
        pltpu.SemaphoreType.DMA,
    ],
    compiler_params=pltpu.CompilerParams(use_tc_tiling_on_sc=False),
)
def _msg_pass(m0, m1, spk0, spk1, rp, agg,
              acc, pbuf, gbuf, dloc, rows, zbuf, rpv, sem):
    c = lax.axis_index("c")
    s = lax.axis_index("s")
    pltpu.sync_copy(rp, rpv)
    ms = (m0, m1)
    spks = (spk0, spk1)

    def _zb(i, carry):
        for g in range(D // 16):
            zbuf[i, pl.ds(g * 16, 16)] = jnp.zeros((16,), jnp.float32)
        return carry
    lax.fori_loop(0, TROWS, _zb, 0)

    def _stripe(sid):
        lo = sid * STRIPE
        pltpu.sync_copy(zbuf, acc.at[pl.ds(s * TROWS, TROWS)])

        @pl.when(s == 0)
        def _():
            pltpu.sync_copy(zbuf.at[pl.ds(0, 8)], acc.at[pl.ds(STRIPE, 8)])
        plsc.subcore_barrier()

        for r in range(2):
            rpvec = rpv[r, pl.ds(sid, 16)]
            p0 = rpvec[0]
            p1 = rpvec[1]
            a0 = (p0 // CH) * CH
            nch = (p1 - a0 + CH - 1) // CH
            nmy = jnp.maximum(nch - s + NS - 1, 0) // NS

            def _fire(t, b, r=r, a0=a0):
                off = a0 + (s + t * NS) * CH
                pltpu.sync_copy(spks[r].at[pl.ds(off, CH)], pbuf.at[b])
                for g in range(CH // 16):
                    v = pbuf[b, pl.ds(g * 16, 16)]
                    gbuf[b, pl.ds(g * 16, 16)] = v & 0xFFFF
                return pltpu.async_copy(ms[r].at[gbuf.at[b]], rows.at[b], sem)

            @pl.when(nmy > 0)
            def _():
                _fire(0, 0)

            def _ch(t, carry, r=r, nmy=nmy, lo=lo):
                b = lax.rem(t, 2)

                @pl.when(t + 1 < nmy)
                def _():
                    _fire(t + 1, 1 - b)
                pltpu.make_async_copy(ms[r].at[gbuf.at[b]], rows.at[b],
                                      sem).wait()
                for g in range(CH // 16):
                    u = (pbuf[b, pl.ds(g * 16, 16)] >> 16) + (32768 - lo)
                    bad = (u < 0) | (u >= STRIPE)
                    dloc[pl.ds(g * 16, 16)] = jnp.where(bad, DUMMY, u)
                pltpu.sync_copy(rows.at[b], acc.at[dloc], add=True)
                return carry
            lax.fori_loop(0, nmy, _ch, 0)

        plsc.subcore_barrier()
        pltpu.sync_copy(acc.at[pl.ds(s * TROWS, TROWS)],
                        agg.at[pl.ds(lo + s * TROWS, TROWS)])

    for k in range(NSTRIPE // NC):
        _stripe(NC * k + c)

    @pl.when(NC * (NSTRIPE // NC) + c < NSTRIPE)
    def _():
        _stripe(NC * (NSTRIPE // NC) + c)


@functools.partial(
    pl.kernel,
    out_type=jax.ShapeDtypeStruct((NC, N_PAD, 8), jnp.float32),
    mesh=_mesh,
    scratch_types=[
        pltpu.VMEM_SHARED((N_PAD, 8), jnp.float32),
        pltpu.VMEM((WORK_CH, CH), jnp.int32),
        pltpu.VMEM((CH, 8), jnp.float32),
        pltpu.SemaphoreType.DMA,
    ],
    compiler_params=pltpu.CompilerParams(use_tc_tiling_on_sc=False),
)
def _attr_agg(attrp, dstp, zeros8, sout, acc, dstv, abuf, sem):
    c = lax.axis_index("c")
    s = lax.axis_index("s")
    w = s * NC + c
    pltpu.sync_copy(dstp.at[w], dstv)
    pltpu.sync_copy(zeros8, acc.at[pl.ds(s * ROWS_PER_TILE, ROWS_PER_TILE)])
    plsc.subcore_barrier()

    def _chunk(j, carry):
        pltpu.sync_copy(attrp.at[pl.ds(w * WORK_E + j * CH, CH)], abuf)
        pltpu.sync_copy(abuf, acc.at[dstv.at[j]], add=True)
        return carry
    lax.fori_loop(0, WORK_CH, _chunk, 0)
    plsc.subcore_barrier()
    pltpu.sync_copy(acc.at[pl.ds(s * ROWS_PER_TILE, ROWS_PER_TILE)],
                    sout.at[c, pl.ds(s * ROWS_PER_TILE, ROWS_PER_TILE)])


def _proj_body(x, win, binp, w0, b0, w1, b1, h, m0, m1):
    hv = jnp.dot(x[...], win[...], preferred_element_type=jnp.float32)
    hv = hv + binp[...]
    h[...] = hv
    m0[...] = jnp.dot(hv, w0[...], preferred_element_type=jnp.float32) + b0[...]
    m1[...] = jnp.dot(hv, w1[...], preferred_element_type=jnp.float32) + b1[...]


def _layer_norm(x, g, b):
    m = jnp.mean(x, axis=-1, keepdims=True)
    d = x - m
    v = jnp.mean(d * d, axis=-1, keepdims=True)
    return g * d * lax.rsqrt(v + 1e-5) + b


def _update_next_body(agg, h, sp, wext, gamma, beta, w0, b0, w1, b1,
                      hout, m0, m1):
    svec = sp[0] + sp[1]
    cterm = jnp.dot(svec, wext[...], preferred_element_type=jnp.float32)
    x = h[...] + jnp.maximum(agg[...] + cterm, 0.0)
    hn = _layer_norm(x, gamma[...], beta[...])
    hout[...] = hn
    m0[...] = jnp.dot(hn, w0[...], preferred_element_type=jnp.float32) + b0[...]
    m1[...] = jnp.dot(hn, w1[...], preferred_element_type=jnp.float32) + b1[...]


def _update_final_body(agg, h, sp, wext, gamma, beta, hout):
    svec = sp[0] + sp[1]
    cterm = jnp.dot(svec, wext[...], preferred_element_type=jnp.float32)
    x = h[...] + jnp.maximum(agg[...] + cterm, 0.0)
    hout[...] = _layer_norm(x, gamma[...], beta[...])


_blk = pl.BlockSpec((NB, D), lambda i: (i, 0))
_wblk = pl.BlockSpec((D, D), lambda i: (0, 0))
_bblk = pl.BlockSpec((1, D), lambda i: (0, 0))
_sblk = pl.BlockSpec((NC, NB, 8), lambda i: (0, i, 0))
_eblk = pl.BlockSpec((8, D), lambda i: (0, 0))
_out = jax.ShapeDtypeStruct((N, D), jnp.float32)

_proj = pl.pallas_call(
    _proj_body,
    grid=(GRID,),
    in_specs=[_blk, _wblk, _bblk, _wblk, _bblk, _wblk, _bblk],
    out_specs=[_blk, _blk, _blk],
    out_shape=[_out, _out, _out],
)

_update_next = pl.pallas_call(
    _update_next_body,
    grid=(GRID,),
    in_specs=[_blk, _blk, _sblk, _eblk, _bblk, _bblk,
              _wblk, _bblk, _wblk, _bblk],
    out_specs=[_blk, _blk, _blk],
    out_shape=[_out, _out, _out],
)

_update_final = pl.pallas_call(
    _update_final_body,
    grid=(GRID,),
    in_specs=[_blk, _blk, _sblk, _eblk, _bblk, _bblk],
    out_specs=_blk,
    out_shape=_out,
)


def kernel(node_feat, edge_index_0, edge_attr_0, edge_index_1, edge_attr_1,
           params):
    f32 = jnp.float32
    x = node_feat.reshape(N, D)
    e = edge_index_0.shape[1]

    bounds = jnp.arange(NSTRIPE + 1, dtype=jnp.int32) * STRIPE

    def _sort_rel(ei):
        packed = ((ei[1] - 32768) << 16) | ei[0]
        spk = lax.sort(packed)
        pbounds = (bounds - 32768) << 16
        rp = jnp.searchsorted(spk, pbounds).astype(jnp.int32)
        spad = E_SRT - e
        spk = jnp.concatenate(
            [spk, jnp.full((spad,), jnp.int32(32767 << 16), jnp.int32)])
        rp = jnp.concatenate(
            [rp, jnp.zeros((48 - NSTRIPE - 1,), jnp.int32)])
        return spk, rp

    spk0, rp0 = _sort_rel(edge_index_0)
    spk1, rp1 = _sort_rel(edge_index_1)
    rp = jnp.stack([rp0, rp1])

    pad = E_PAD - e
    dstp = jnp.concatenate(
        [edge_index_0[1], jnp.full((pad,), N, jnp.int32)]
    ).reshape(NW, WORK_CH, CH)
    attrp = jnp.concatenate([
        jnp.concatenate(
            [edge_attr_0, jnp.ones((e, 1), f32), jnp.zeros((e, 3), f32)], 1),
        jnp.zeros((pad, 8), f32)], axis=0)
    zeros8 = jnp.zeros((ROWS_PER_TILE, 8), f32)

    sp = _attr_agg(attrp, dstp, zeros8)

    p = params
    l0, l1 = p["layers"]
    wext = [jnp.concatenate(
        [l["edge_W"][0], l["edge_b"][0][None, :], jnp.zeros((3, D), f32)], 0)
        for l in (l0, l1)]

    h, m0, m1 = _proj(
        x, p["input_W"], p["input_b"][None], l0["node_W"][0],
        l0["node_b"][0][None], l0["node_W"][1], l0["node_b"][1][None])
    agg = _msg_pass(m0, m1, spk0, spk1, rp)
    h, m0, m1 = _update_next(
        agg, h, sp, wext[0], l0["gamma"][None], l0["beta"][None],
        l1["node_W"][0], l1["node_b"][0][None],
        l1["node_W"][1], l1["node_b"][1][None])
    agg = _msg_pass(m0, m1, spk0, spk1, rp)
    h = _update_final(
        agg, h, sp, wext[1], l1["gamma"][None], l1["beta"][None])
    return h.reshape(1, N, D)
